# Initial kernel scaffold; baseline (speedup 1.0000x reference)
import jax
import jax.numpy as jnp
from jax.experimental import pallas as pl


def kernel(xt, dataset):
    raise NotImplementedError("write your pallas kernel here")


# speedup vs baseline: 2.0670x; 2.0670x over previous
import functools

import jax
import jax.numpy as jnp
from jax import lax
from jax.experimental import pallas as pl
from jax.experimental.pallas import tpu as pltpu

_K = 20
_BIG = 1e30
_SUB = 256


def _sigmoid(t):
    return jnp.where(t >= 0.0, 1.0 / (1.0 + jnp.exp(-t)), jnp.exp(t) / (1.0 + jnp.exp(t)))


def _body(nchunks, chunk, xt_ref, dt_ref, dsq_ref, out_ref,
          d2_ref, rv_ref, rx_ref, ry_ref, rz_ref):
    qb = xt_ref.shape[0]
    nsub = chunk // _SUB
    xq = xt_ref[...]
    qx = xq[:, 0:1]
    qy = xq[:, 1:2]
    qz = xq[:, 2:3]
    q2 = jnp.sum(xq * xq, axis=1, keepdims=True)

    rv_ref[...] = jnp.full((qb, _K), _BIG, jnp.float32)
    rx_ref[...] = jnp.zeros((qb, _K), jnp.float32)
    ry_ref[...] = jnp.zeros((qb, _K), jnp.float32)
    rz_ref[...] = jnp.zeros((qb, _K), jnp.float32)

    iota_s = lax.broadcasted_iota(jnp.int32, (qb, _SUB), 1)
    iota_k = lax.broadcasted_iota(jnp.int32, (qb, _K), 1)

    def chunk_body(c, _):
        dchunk = dt_ref[:, pl.ds(c * chunk, chunk)]
        dsqc = dsq_ref[0:1, pl.ds(c * chunk, chunk)]
        qp = lax.dot_general(xq, dchunk, (((1,), (0,)), ((), ())),
                             preferred_element_type=jnp.float32)
        d2_ref[...] = (q2 - 2.0 * qp) + dsqc

        def sub_body(s, _2):
            dsub = dt_ref[:, pl.ds(c * chunk + s * _SUB, _SUB)]

            def cond(go):
                return go

            def merge(_go):
                d2 = d2_ref[:, pl.ds(s * _SUB, _SUB)]
                cmin = jnp.min(d2, axis=1, keepdims=True)
                rv = rv_ref[...]
                rmax = jnp.max(rv, axis=1, keepdims=True)
                improve = cmin < rmax
                eq = d2 == cmin
                colidx = jnp.min(jnp.where(eq, iota_s, jnp.int32(0x7FFFFFFF)),
                                 axis=1, keepdims=True)
                onehot = (iota_s == colidx) & improve
                sel = jnp.where(onehot, 1.0, 0.0)
                cxyz = lax.dot_general(sel, dsub, (((1,), (1,)), ((), ())),
                                       preferred_element_type=jnp.float32)
                ccx = cxyz[:, 0:1] - qx
                ccy = cxyz[:, 1:2] - qy
                ccz = cxyz[:, 2:3] - qz
                seq = rv == rmax
                scol = jnp.min(jnp.where(seq, iota_k, jnp.int32(0x7FFFFFFF)),
                               axis=1, keepdims=True)
                sone = (iota_k == scol) & improve
                rv_new = jnp.where(sone, cmin, rv)
                rv_ref[...] = rv_new
                rx_ref[...] = jnp.where(sone, ccx, rx_ref[...])
                ry_ref[...] = jnp.where(sone, ccy, ry_ref[...])
                rz_ref[...] = jnp.where(sone, ccz, rz_ref[...])
                d2_new = jnp.where(onehot, _BIG, d2)
                d2_ref[:, pl.ds(s * _SUB, _SUB)] = d2_new
                cmin2 = jnp.min(d2_new, axis=1, keepdims=True)
                rmax2 = jnp.max(rv_new, axis=1, keepdims=True)
                return jnp.any(cmin2 < rmax2)

            go0 = jnp.any(
                jnp.min(d2_ref[:, pl.ds(s * _SUB, _SUB)], axis=1, keepdims=True)
                < jnp.max(rv_ref[...], axis=1, keepdims=True))
            lax.while_loop(cond, merge, go0)
            return 0

        lax.fori_loop(0, nsub, sub_body, 0)
        return 0

    lax.fori_loop(0, nchunks, chunk_body, 0)

    cx = rx_ref[...]
    cy = ry_ref[...]
    cz = rz_ref[...]
    m11 = jnp.sum(cx * cx, axis=1, keepdims=True)
    m12 = jnp.sum(cx * cy, axis=1, keepdims=True)
    m13 = jnp.sum(cx, axis=1, keepdims=True)
    m22 = jnp.sum(cy * cy, axis=1, keepdims=True)
    m23 = jnp.sum(cy, axis=1, keepdims=True)
    m33 = jnp.full_like(m13, float(_K))
    r1 = jnp.sum(cx * cz, axis=1, keepdims=True)
    r2 = jnp.sum(cy * cz, axis=1, keepdims=True)
    r3 = jnp.sum(cz, axis=1, keepdims=True)
    c11 = m22 * m33 - m23 * m23
    c12 = m13 * m23 - m12 * m33
    c13 = m12 * m23 - m13 * m22
    c22 = m11 * m33 - m13 * m13
    c23 = m12 * m13 - m11 * m23
    c33 = m11 * m22 - m12 * m12
    det = m11 * c11 + m12 * c12 + m13 * c13
    inv_det = 1.0 / det
    a = (c11 * r1 + c12 * r2 + c13 * r3) * inv_det
    b = (c12 * r1 + c22 * r2 + c23 * r3) * inv_det
    cconst = (c13 * r1 + c23 * r2 + c33 * r3) * inv_det
    nn = a * a + b * b + 1.0
    s = cconst / nn
    closeness = cconst * s
    height = jnp.exp(qz + s)

    def pen(v):
        return _sigmoid((v - 5.0) / 0.1) + 1.0 - _sigmoid((v + 5.0) / 0.1)

    boundary = pen(qx) + pen(qy)
    total = closeness + height + boundary
    out_ref[...] = total[:, 0].reshape(1, 1, qb)


def kernel(xt, dataset):
    n = xt.shape[0]
    dsize = dataset.shape[0]
    chunk = 2048 if dsize >= 2048 else _SUB
    pad = (-dsize) % chunk
    psize = dsize + pad
    nchunks = psize // chunk
    qb = 256 if n % 256 == 0 else n
    nblocks = n // qb

    dpad = jnp.concatenate(
        [dataset, jnp.zeros((pad, 3), jnp.float32)], axis=0) if pad else dataset
    dt = dpad.T
    dsq = jnp.sum(dpad * dpad, axis=1)
    if pad:
        dsq = dsq.at[dsize:].set(_BIG)
    dsq = dsq.reshape(1, psize)

    out = pl.pallas_call(
        functools.partial(_body, nchunks, chunk),
        grid=(nblocks,),
        in_specs=[
            pl.BlockSpec((qb, 3), lambda i: (i, 0)),
            pl.BlockSpec((3, psize), lambda i: (0, 0)),
            pl.BlockSpec((1, psize), lambda i: (0, 0)),
        ],
        out_specs=pl.BlockSpec((1, 1, qb), lambda i: (i, 0, 0)),
        out_shape=jax.ShapeDtypeStruct((nblocks, 1, qb), jnp.float32),
        scratch_shapes=[
            pltpu.VMEM((qb, chunk), jnp.float32),
            pltpu.VMEM((qb, _K), jnp.float32),
            pltpu.VMEM((qb, _K), jnp.float32),
            pltpu.VMEM((qb, _K), jnp.float32),
            pltpu.VMEM((qb, _K), jnp.float32),
        ],
    )(xt, dt, dsq)
    return out.reshape(n)
